# Initial kernel scaffold; baseline (speedup 1.0000x reference)
import jax
import jax.numpy as jnp
from jax.experimental import pallas as pl


def kernel(xyz, features, W1, b1, W2, b2):
    raise NotImplementedError("write your pallas kernel here")


# speedup vs baseline: 254.3599x; 254.3599x over previous
import functools

import jax
import jax.numpy as jnp
import numpy as np
from jax import lax
from jax.experimental import pallas as pl
from jax.experimental.pallas import tpu as pltpu
from jax.experimental.pallas import tpu_sc as plsc

_B = 8
_N = 4096
_C = 64
_M = 1024
_S = 32
_R2 = np.float32(0.2 * 0.2)
_D = 80
_NW = 32
_MPW = (_B * _M) // _NW
_ROWS = _B * _M * _S
_RPW = _ROWS // _NW
_CH = 128
_NCH = _RPW // _CH


def _fps_body(xs_ref, ys_ref, zs_ref, nx_ref, ny_ref, nz_ref):
    B, N = xs_ref.shape
    M = nx_ref.shape[1]
    colN = lax.broadcasted_iota(jnp.int32, (B, N), 1)
    colM = lax.broadcasted_iota(jnp.int32, (B, M), 1)
    xs = xs_ref[...]
    ys = ys_ref[...]
    zs = zs_ref[...]

    def body(i, carry):
        dists, far = carry
        sel = colN == far
        cx = jnp.sum(jnp.where(sel, xs, 0.0), axis=1, keepdims=True)
        cy = jnp.sum(jnp.where(sel, ys, 0.0), axis=1, keepdims=True)
        cz = jnp.sum(jnp.where(sel, zs, 0.0), axis=1, keepdims=True)
        outm = colM == i
        nx_ref[...] = jnp.where(outm, cx, nx_ref[...])
        ny_ref[...] = jnp.where(outm, cy, ny_ref[...])
        nz_ref[...] = jnp.where(outm, cz, nz_ref[...])
        dx = xs - cx
        dy = ys - cy
        dz = zs - cz
        d = dx * dx + dy * dy + dz * dz
        dists = jnp.minimum(dists, d)
        mx = jnp.max(dists, axis=1, keepdims=True)
        far = jnp.min(jnp.where(dists == mx, colN, N), axis=1, keepdims=True)
        return dists, far

    dists0 = jnp.full((B, N), 1e10, dtype=jnp.float32)
    far0 = jnp.zeros((B, 1), dtype=jnp.int32)
    lax.fori_loop(0, M, body, (dists0, far0))


def _fps(xs, ys, zs):
    B = xs.shape[0]
    shp = jax.ShapeDtypeStruct((B, _M), jnp.float32)
    return pl.pallas_call(
        _fps_body,
        out_shape=[shp, shp, shp],
    )(xs, ys, zs)


def _rnd16(v):
    u = plsc.bitcast(v, jnp.uint32)
    r = (u + jnp.uint32(0x7FFF) + ((u >> jnp.uint32(16)) & jnp.uint32(1))) & jnp.uint32(0xFFFF0000)
    return plsc.bitcast(r, jnp.float32)


def _ballq_body(xs_hbm, ys_hbm, zs_hbm,
                nx_hbm, ny_hbm, nz_hbm, out_hbm,
                x_v, y_v, z_v, b2_v, nx_v, ny_v, nz_v, o_v):
    cid = lax.axis_index("c")
    sid = lax.axis_index("s")
    wid = sid * 2 + cid
    b = wid // 4
    mseg = (wid % 4) * _MPW

    gm = b * _M + mseg
    pltpu.sync_copy(xs_hbm.at[pl.ds(b * _N, _N)], x_v)
    pltpu.sync_copy(ys_hbm.at[pl.ds(b * _N, _N)], y_v)
    pltpu.sync_copy(zs_hbm.at[pl.ds(b * _N, _N)], z_v)
    pltpu.sync_copy(nx_hbm.at[pl.ds(gm, _MPW)], nx_v.at[pl.ds(0, _MPW)])
    pltpu.sync_copy(ny_hbm.at[pl.ds(gm, _MPW)], ny_v.at[pl.ds(0, _MPW)])
    pltpu.sync_copy(nz_hbm.at[pl.ds(gm, _MPW)], nz_v.at[pl.ds(0, _MPW)])

    lanes = lax.iota(jnp.int32, 16)
    nbase = b * _N

    def b2fill(j, _):
        n = j * 16
        xv = x_v[pl.ds(n, 16)]
        yv = y_v[pl.ds(n, 16)]
        zv = z_v[pl.ds(n, 16)]
        b2_v[pl.ds(n, 16)] = xv * xv + yv * yv + zv * zv
        return 0
    lax.fori_loop(0, _N // 16, b2fill, 0)

    def bround(j, _):
        n = j * 16
        x_v[pl.ds(n, 16)] = _rnd16(x_v[pl.ds(n, 16)])
        y_v[pl.ds(n, 16)] = _rnd16(y_v[pl.ds(n, 16)])
        z_v[pl.ds(n, 16)] = _rnd16(z_v[pl.ds(n, 16)])
        return 0
    lax.fori_loop(0, _N // 16, bround, 0)

    def per_m(m, _):
        cxv = nx_v[pl.ds(m, 16)]
        cyv = ny_v[pl.ds(m, 16)]
        czv = nz_v[pl.ds(m, 16)]
        cx = cxv[0]
        cy = cyv[0]
        cz = czv[0]
        cxb = _rnd16(cxv)[0]
        cyb = _rnd16(cyv)[0]
        czb = _rnd16(czv)[0]
        a2 = cx * cx + cy * cy + cz * cz

        def step(j, cnt):
            n = j * 16
            xv = x_v[pl.ds(n, 16)]
            yv = y_v[pl.ds(n, 16)]
            zv = z_v[pl.ds(n, 16)]
            b2v = b2_v[pl.ds(n, 16)]
            ab = xv * cxb + yv * cyb + zv * czb
            d2 = a2 + b2v - 2.0 * ab
            msk = d2 < _R2
            mi = msk.astype(jnp.int32)
            pos = plsc.cumsum(mi)
            offs = m * _S + cnt + pos - 1
            vals = (nbase + n) + lanes
            plsc.store_scatter(o_v, [offs], vals, mask=msk)
            return cnt + jnp.sum(mi)

        def outer(k, cnt):
            return lax.cond(
                cnt < _S,
                lambda c: lax.fori_loop(k * 16, (k + 1) * 16, step, c),
                lambda c: c,
                cnt)

        cnt_f = lax.fori_loop(0, _N // 256, outer, jnp.int32(0))

        first = o_v[pl.ds(m * _S, 16)][0]
        first = jnp.where(cnt_f > 0, first, nbase)
        for h in range(_S // 16):
            posh = lanes + h * 16
            cur = plsc.load_gather(o_v, [m * _S + posh])
            outv = jnp.where(posh < cnt_f, cur, first)
            plsc.store_scatter(o_v, [m * _S + posh], outv)
        return 0

    lax.fori_loop(0, _MPW, per_m, 0)
    pltpu.sync_copy(o_v.at[pl.ds(0, _MPW * _S)],
                    out_hbm.at[pl.ds(wid * _MPW * _S, _MPW * _S)])


def _ballq(xs, ys, zs, nx, ny, nz):
    mesh = plsc.VectorSubcoreMesh(core_axis_name="c", subcore_axis_name="s")
    f = functools.partial(
        pl.kernel,
        mesh=mesh,
        out_type=jax.ShapeDtypeStruct((_ROWS,), jnp.int32),
        scratch_types=[
            pltpu.VMEM((_N,), jnp.float32),
            pltpu.VMEM((_N,), jnp.float32),
            pltpu.VMEM((_N,), jnp.float32),
            pltpu.VMEM((_N,), jnp.float32),
            pltpu.VMEM((_MPW + 16,), jnp.float32),
            pltpu.VMEM((_MPW + 16,), jnp.float32),
            pltpu.VMEM((_MPW + 16,), jnp.float32),
            pltpu.VMEM((_MPW * _S + 288,), jnp.int32),
        ],
        compiler_params=pltpu.CompilerParams(
            needs_layout_passes=False, use_tc_tiling_on_sc=False),
    )(_ballq_body)
    return f(xs, ys, zs, nx, ny, nz)


def _gather_body(table_hbm, idx_hbm, out_hbm, idx_v, rows_v, sem):
    cid = lax.axis_index("c")
    sid = lax.axis_index("s")
    wid = sid * 2 + cid
    base = wid * _RPW

    pltpu.sync_copy(idx_hbm.at[wid], idx_v)

    def chunk(j, _):
        pltpu.async_copy(table_hbm.at[idx_v.at[j]], rows_v, sem).wait()
        pltpu.sync_copy(rows_v, out_hbm.at[pl.ds(base + j * _CH, _CH)])
        return 0

    lax.fori_loop(0, _NCH, chunk, 0)


def _gather(table, idx3):
    mesh = plsc.VectorSubcoreMesh(core_axis_name="c", subcore_axis_name="s")
    f = functools.partial(
        pl.kernel,
        mesh=mesh,
        out_type=jax.ShapeDtypeStruct((_ROWS, _D), jnp.float32),
        scratch_types=[
            pltpu.VMEM((_NCH, _CH), jnp.int32),
            pltpu.VMEM((_CH, _D), jnp.float32),
            pltpu.SemaphoreType.DMA,
        ],
        compiler_params=pltpu.CompilerParams(use_tc_tiling_on_sc=False),
    )(_gather_body)
    return f(table, idx3)


_BLK_M = 128
_BLK_R = _BLK_M * _S


def _mlp_body(g_ref, nx_ref, w1_ref, w1x_ref, b1_ref, w2_ref, b2_ref, o_ref):
    g = g_ref[...]
    h1 = jnp.dot(g, w1_ref[...], preferred_element_type=jnp.float32)
    corr = jnp.dot(nx_ref[...], w1x_ref[...], preferred_element_type=jnp.float32)
    h1 = jnp.maximum(h1 - corr + b1_ref[...], 0.0)
    h2 = jnp.dot(h1, w2_ref[...], preferred_element_type=jnp.float32) + b2_ref[...]
    h2 = jnp.maximum(h2, 0.0)
    o_ref[...] = jnp.max(h2.reshape(_BLK_M, _S, 128), axis=1)


def _mlp(g, nrep, w1p, w1x, b1, w2t, b2):
    grid = _ROWS // _BLK_R
    return pl.pallas_call(
        _mlp_body,
        grid=(grid,),
        in_specs=[
            pl.BlockSpec((_BLK_R, _D), lambda i: (i, 0)),
            pl.BlockSpec((_BLK_R, 3), lambda i: (i, 0)),
            pl.BlockSpec((_D, 64), lambda i: (0, 0)),
            pl.BlockSpec((3, 64), lambda i: (0, 0)),
            pl.BlockSpec((1, 64), lambda i: (0, 0)),
            pl.BlockSpec((64, 128), lambda i: (0, 0)),
            pl.BlockSpec((1, 128), lambda i: (0, 0)),
        ],
        out_specs=pl.BlockSpec((_BLK_M, 128), lambda i: (i, 0)),
        out_shape=jax.ShapeDtypeStruct((_B * _M, 128), jnp.float32),
    )(g, nrep, w1p, w1x, b1, w2t, b2)


def kernel(xyz, features, W1, b1, W2, b2):
    B, N, _ = xyz.shape
    xs = xyz[:, :, 0]
    ys = xyz[:, :, 1]
    zs = xyz[:, :, 2]

    nx, ny, nz = _fps(xs, ys, zs)

    idx = _ballq(xs.reshape(-1), ys.reshape(-1), zs.reshape(-1),
                 nx.reshape(-1), ny.reshape(-1), nz.reshape(-1))
    idx3 = idx.reshape(_NW, _NCH, _CH)

    table = jnp.concatenate(
        [xyz, jnp.transpose(features, (0, 2, 1)),
         jnp.zeros((B, N, _D - 3 - _C), jnp.float32)], axis=-1
    ).reshape(B * N, _D)
    g = _gather(table, idx3)

    new_xyz = jnp.stack([nx, ny, nz], axis=-1)
    nrep = jnp.repeat(new_xyz.reshape(B * _M, 3), _S, axis=0)

    w1p = jnp.concatenate(
        [W1, jnp.zeros((64, _D - 3 - _C), jnp.float32)], axis=1).T
    feats = _mlp(g, nrep, w1p, w1p[:3], b1[None, :], W2.T, b2[None, :])

    new_features = feats.reshape(B, _M, 128).transpose(0, 2, 1)
    return new_xyz, new_features


# speedup vs baseline: 278.4677x; 1.0948x over previous
import functools

import jax
import jax.numpy as jnp
import numpy as np
from jax import lax
from jax.experimental import pallas as pl
from jax.experimental.pallas import tpu as pltpu
from jax.experimental.pallas import tpu_sc as plsc

_B = 8
_N = 4096
_C = 64
_M = 1024
_S = 32
_R2 = np.float32(0.2 * 0.2)
_D = 80
_NW = 32
_MPW = (_B * _M) // _NW
_ROWS = _B * _M * _S
_RPW = _ROWS // _NW
_CH = 128
_NCH = _RPW // _CH


def _fps_body(xs_ref, ys_ref, zs_ref, nx_ref, ny_ref, nz_ref):
    B, N = xs_ref.shape
    M = nx_ref.shape[1]
    colN = lax.broadcasted_iota(jnp.int32, (B, N), 1)
    colM = lax.broadcasted_iota(jnp.int32, (B, M), 1)
    xs = xs_ref[...]
    ys = ys_ref[...]
    zs = zs_ref[...]

    def body(i, carry):
        dists, far = carry
        sel = colN == far
        cx = jnp.sum(jnp.where(sel, xs, 0.0), axis=1, keepdims=True)
        cy = jnp.sum(jnp.where(sel, ys, 0.0), axis=1, keepdims=True)
        cz = jnp.sum(jnp.where(sel, zs, 0.0), axis=1, keepdims=True)
        outm = colM == i
        nx_ref[...] = jnp.where(outm, cx, nx_ref[...])
        ny_ref[...] = jnp.where(outm, cy, ny_ref[...])
        nz_ref[...] = jnp.where(outm, cz, nz_ref[...])
        dx = xs - cx
        dy = ys - cy
        dz = zs - cz
        d = dx * dx + dy * dy + dz * dz
        dists = jnp.minimum(dists, d)
        mx = jnp.max(dists, axis=1, keepdims=True)
        far = jnp.min(jnp.where(dists == mx, colN, N), axis=1, keepdims=True)
        return dists, far

    dists0 = jnp.full((B, N), 1e10, dtype=jnp.float32)
    far0 = jnp.zeros((B, 1), dtype=jnp.int32)
    lax.fori_loop(0, M, body, (dists0, far0))


def _fps(xs, ys, zs):
    B = xs.shape[0]
    shp = jax.ShapeDtypeStruct((B, _M), jnp.float32)
    return pl.pallas_call(
        _fps_body,
        out_shape=[shp, shp, shp],
    )(xs, ys, zs)


def _rnd16(v):
    u = plsc.bitcast(v, jnp.uint32)
    r = (u + jnp.uint32(0x7FFF) + ((u >> jnp.uint32(16)) & jnp.uint32(1))) & jnp.uint32(0xFFFF0000)
    return plsc.bitcast(r, jnp.float32)


def _ballq_body(xs_hbm, ys_hbm, zs_hbm,
                nx_hbm, ny_hbm, nz_hbm, out_hbm,
                x_v, y_v, z_v, b2_v, nx_v, ny_v, nz_v, o_v):
    cid = lax.axis_index("c")
    sid = lax.axis_index("s")
    wid = sid * 2 + cid
    b = wid // 4
    mseg = (wid % 4) * _MPW

    gm = b * _M + mseg
    pltpu.sync_copy(xs_hbm.at[pl.ds(b * _N, _N)], x_v)
    pltpu.sync_copy(ys_hbm.at[pl.ds(b * _N, _N)], y_v)
    pltpu.sync_copy(zs_hbm.at[pl.ds(b * _N, _N)], z_v)
    pltpu.sync_copy(nx_hbm.at[pl.ds(gm, _MPW)], nx_v.at[pl.ds(0, _MPW)])
    pltpu.sync_copy(ny_hbm.at[pl.ds(gm, _MPW)], ny_v.at[pl.ds(0, _MPW)])
    pltpu.sync_copy(nz_hbm.at[pl.ds(gm, _MPW)], nz_v.at[pl.ds(0, _MPW)])

    lanes = lax.iota(jnp.int32, 16)
    nbase = b * _N

    def b2fill(j, _):
        n = j * 16
        xv = x_v[pl.ds(n, 16)]
        yv = y_v[pl.ds(n, 16)]
        zv = z_v[pl.ds(n, 16)]
        b2_v[pl.ds(n, 16)] = xv * xv + yv * yv + zv * zv
        return 0
    lax.fori_loop(0, _N // 16, b2fill, 0)

    def bround(j, _):
        n = j * 16
        x_v[pl.ds(n, 16)] = _rnd16(x_v[pl.ds(n, 16)])
        y_v[pl.ds(n, 16)] = _rnd16(y_v[pl.ds(n, 16)])
        z_v[pl.ds(n, 16)] = _rnd16(z_v[pl.ds(n, 16)])
        return 0
    lax.fori_loop(0, _N // 16, bround, 0)

    def per_m(m, _):
        cxv = nx_v[pl.ds(m, 16)]
        cyv = ny_v[pl.ds(m, 16)]
        czv = nz_v[pl.ds(m, 16)]
        cx = cxv[0]
        cy = cyv[0]
        cz = czv[0]
        cxb = _rnd16(cxv)[0]
        cyb = _rnd16(cyv)[0]
        czb = _rnd16(czv)[0]
        a2 = cx * cx + cy * cy + cz * cz

        def step(j, cnt):
            n = j * 16
            xv = x_v[pl.ds(n, 16)]
            yv = y_v[pl.ds(n, 16)]
            zv = z_v[pl.ds(n, 16)]
            b2v = b2_v[pl.ds(n, 16)]
            ab = xv * cxb + yv * cyb + zv * czb
            d2 = a2 + b2v - 2.0 * ab
            msk = d2 < _R2
            vals = (nbase + n) + lanes
            plsc.store_compressed(o_v.at[pl.ds(m * _S + cnt, 16)], vals,
                                  mask=msk)
            return cnt + plsc.all_reduce_population_count(msk)[0]

        def outer(k, cnt):
            return lax.cond(
                cnt < _S,
                lambda c: lax.fori_loop(k * 16, (k + 1) * 16, step, c),
                lambda c: c,
                cnt)

        cnt_f = lax.fori_loop(0, _N // 256, outer, jnp.int32(0))

        first = o_v[pl.ds(m * _S, 16)][0]
        first = jnp.where(cnt_f > 0, first, nbase)
        for h in range(_S // 16):
            posh = lanes + h * 16
            cur = plsc.load_gather(o_v, [m * _S + posh])
            outv = jnp.where(posh < cnt_f, cur, first)
            plsc.store_scatter(o_v, [m * _S + posh], outv)
        return 0

    lax.fori_loop(0, _MPW, per_m, 0)
    pltpu.sync_copy(o_v.at[pl.ds(0, _MPW * _S)],
                    out_hbm.at[pl.ds(wid * _MPW * _S, _MPW * _S)])


def _ballq(xs, ys, zs, nx, ny, nz):
    mesh = plsc.VectorSubcoreMesh(core_axis_name="c", subcore_axis_name="s")
    f = functools.partial(
        pl.kernel,
        mesh=mesh,
        out_type=jax.ShapeDtypeStruct((_ROWS,), jnp.int32),
        scratch_types=[
            pltpu.VMEM((_N,), jnp.float32),
            pltpu.VMEM((_N,), jnp.float32),
            pltpu.VMEM((_N,), jnp.float32),
            pltpu.VMEM((_N,), jnp.float32),
            pltpu.VMEM((_MPW + 16,), jnp.float32),
            pltpu.VMEM((_MPW + 16,), jnp.float32),
            pltpu.VMEM((_MPW + 16,), jnp.float32),
            pltpu.VMEM((_MPW * _S + 288,), jnp.int32),
        ],
        compiler_params=pltpu.CompilerParams(
            needs_layout_passes=False, use_tc_tiling_on_sc=False),
    )(_ballq_body)
    return f(xs, ys, zs, nx, ny, nz)


def _gather_body(table_hbm, idx_hbm, out_hbm, idx_v, rows_v, sem):
    cid = lax.axis_index("c")
    sid = lax.axis_index("s")
    wid = sid * 2 + cid
    base = wid * _RPW

    pltpu.sync_copy(idx_hbm.at[wid], idx_v)

    def chunk(j, _):
        pltpu.async_copy(table_hbm.at[idx_v.at[j]], rows_v, sem).wait()
        pltpu.sync_copy(rows_v, out_hbm.at[pl.ds(base + j * _CH, _CH)])
        return 0

    lax.fori_loop(0, _NCH, chunk, 0)


def _gather(table, idx3):
    mesh = plsc.VectorSubcoreMesh(core_axis_name="c", subcore_axis_name="s")
    f = functools.partial(
        pl.kernel,
        mesh=mesh,
        out_type=jax.ShapeDtypeStruct((_ROWS, _D), jnp.float32),
        scratch_types=[
            pltpu.VMEM((_NCH, _CH), jnp.int32),
            pltpu.VMEM((_CH, _D), jnp.float32),
            pltpu.SemaphoreType.DMA,
        ],
        compiler_params=pltpu.CompilerParams(use_tc_tiling_on_sc=False),
    )(_gather_body)
    return f(table, idx3)


_BLK_M = 128
_BLK_R = _BLK_M * _S


def _mlp_body(g_ref, nx_ref, w1_ref, w1x_ref, b1_ref, w2_ref, b2_ref, o_ref):
    g = g_ref[...]
    h1 = jnp.dot(g, w1_ref[...], preferred_element_type=jnp.float32)
    corr = jnp.dot(nx_ref[...], w1x_ref[...], preferred_element_type=jnp.float32)
    h1 = jnp.maximum(h1 - corr + b1_ref[...], 0.0)
    h2 = jnp.dot(h1, w2_ref[...], preferred_element_type=jnp.float32) + b2_ref[...]
    h2 = jnp.maximum(h2, 0.0)
    o_ref[...] = jnp.max(h2.reshape(_BLK_M, _S, 128), axis=1)


def _mlp(g, nrep, w1p, w1x, b1, w2t, b2):
    grid = _ROWS // _BLK_R
    return pl.pallas_call(
        _mlp_body,
        grid=(grid,),
        in_specs=[
            pl.BlockSpec((_BLK_R, _D), lambda i: (i, 0)),
            pl.BlockSpec((_BLK_R, 3), lambda i: (i, 0)),
            pl.BlockSpec((_D, 64), lambda i: (0, 0)),
            pl.BlockSpec((3, 64), lambda i: (0, 0)),
            pl.BlockSpec((1, 64), lambda i: (0, 0)),
            pl.BlockSpec((64, 128), lambda i: (0, 0)),
            pl.BlockSpec((1, 128), lambda i: (0, 0)),
        ],
        out_specs=pl.BlockSpec((_BLK_M, 128), lambda i: (i, 0)),
        out_shape=jax.ShapeDtypeStruct((_B * _M, 128), jnp.float32),
    )(g, nrep, w1p, w1x, b1, w2t, b2)


def kernel(xyz, features, W1, b1, W2, b2):
    B, N, _ = xyz.shape
    xs = xyz[:, :, 0]
    ys = xyz[:, :, 1]
    zs = xyz[:, :, 2]

    nx, ny, nz = _fps(xs, ys, zs)

    idx = _ballq(xs.reshape(-1), ys.reshape(-1), zs.reshape(-1),
                 nx.reshape(-1), ny.reshape(-1), nz.reshape(-1))
    idx3 = idx.reshape(_NW, _NCH, _CH)

    table = jnp.concatenate(
        [xyz, jnp.transpose(features, (0, 2, 1)),
         jnp.zeros((B, N, _D - 3 - _C), jnp.float32)], axis=-1
    ).reshape(B * N, _D)
    g = _gather(table, idx3)

    new_xyz = jnp.stack([nx, ny, nz], axis=-1)
    nrep = jnp.repeat(new_xyz.reshape(B * _M, 3), _S, axis=0)

    w1p = jnp.concatenate(
        [W1, jnp.zeros((64, _D - 3 - _C), jnp.float32)], axis=1).T
    feats = _mlp(g, nrep, w1p, w1p[:3], b1[None, :], W2.T, b2[None, :])

    new_features = feats.reshape(B, _M, 128).transpose(0, 2, 1)
    return new_xyz, new_features
